# baseline (speedup 1.0000x reference)
import functools

import jax
import jax.numpy as jnp
from jax import lax
from jax.experimental import pallas as pl
from jax.experimental.pallas import tpu as pltpu
from jax.experimental.pallas import tpu_sc as plsc

N = 10000
E = 320000
INC = 128
HID = 32
OUT = 128

NC = 2
NS = 16
NW = NC * NS
CHUNK = 128
NCH = E // CHUNK
CPW = NCH // NW
NEXTRA = NCH - CPW * NW
NP = 10240
RPT = NP // NS

BLK = 10000


def _proj_body(x_ref, wl_ref, wr_ref, bl_ref, y_ref, z_ref):
    x = x_ref[...]
    y_ref[...] = lax.dot_general(x, wl_ref[...], (((1,), (1,)), ((), ())),
                                 preferred_element_type=jnp.float32)
    z_ref[...] = lax.dot_general(x, wr_ref[...], (((1,), (1,)), ((), ())),
                                 preferred_element_type=jnp.float32) + bl_ref[...]


def _proj(x, wl, wr, bl):
    return pl.pallas_call(
        _proj_body,
        grid=(N // BLK,),
        in_specs=[
            pl.BlockSpec((BLK, INC), lambda i: (i, 0)),
            pl.BlockSpec(memory_space=pltpu.VMEM),
            pl.BlockSpec(memory_space=pltpu.VMEM),
            pl.BlockSpec(memory_space=pltpu.VMEM),
        ],
        out_specs=[
            pl.BlockSpec((BLK, HID), lambda i: (i, 0)),
            pl.BlockSpec((BLK, HID), lambda i: (i, 0)),
        ],
        out_shape=[
            jax.ShapeDtypeStruct((N, HID), jnp.float32),
            jax.ShapeDtypeStruct((N, HID), jnp.float32),
        ],
    )(x, wl, wr, bl)


NBUF = 6


def _seg_body(with_deg, feat, eidx, zrow, zdeg, *refs):
    if with_deg:
        (out, deg_out, src_v, dst_v, rows_v, ones_v, acc_s, dacc_s,
         dsem, *sems) = refs
    else:
        out, src_v, dst_v, rows_v, acc_s, *sems = refs
    gsem = sems[:NBUF]
    ssem = sems[NBUF:]

    c = lax.axis_index("c")
    s = lax.axis_index("s")
    w = c * NS + s

    pltpu.sync_copy(zrow.at[pl.ds(s * RPT, RPT), :],
                    acc_s.at[pl.ds(s * RPT, RPT), :])
    if with_deg:
        pltpu.sync_copy(zdeg.at[pl.ds(s * RPT, RPT)],
                        dacc_s.at[pl.ds(s * RPT, RPT)])
        for i in range(CHUNK // 16):
            ones_v[pl.ds(i * 16, 16)] = jnp.ones((16,), jnp.float32)

    plsc.subcore_barrier()

    base = CPW * w + jnp.minimum(w, NEXTRA)

    pltpu.sync_copy(eidx.at[0, pl.ds(base, CPW), :],
                    src_v.at[pl.ds(0, CPW), :])
    pltpu.sync_copy(eidx.at[1, pl.ds(base, CPW), :],
                    dst_v.at[pl.ds(0, CPW), :])

    for b in range(NBUF):
        pltpu.async_copy(feat.at[src_v.at[b]], rows_v.at[b], gsem[b])

    niter = CPW // NBUF

    @pl.loop(0, niter)
    def _grp(g):
        jb = g * NBUF
        degs = []
        for b in range(NBUF):
            j = jb + b
            pltpu.make_async_copy(feat.at[src_v.at[j]], rows_v.at[b],
                                  gsem[b]).wait()
            sc = pltpu.async_copy(rows_v.at[b], acc_s.at[dst_v.at[j]],
                                  ssem[b], add=True)
            if with_deg:
                degs.append(pltpu.async_copy(ones_v,
                                             dacc_s.at[dst_v.at[j]],
                                             dsem, add=True))
            sc.wait()
            @pl.when(g < niter - 1)
            def _refill():
                pltpu.async_copy(feat.at[src_v.at[j + NBUF]],
                                 rows_v.at[b], gsem[b])
        for d in degs:
            d.wait()

    @pl.when(w < NEXTRA)
    def _extra():
        pltpu.sync_copy(eidx.at[0, pl.ds(base + CPW, 1), :],
                        src_v.at[pl.ds(CPW, 1), :])
        pltpu.sync_copy(eidx.at[1, pl.ds(base + CPW, 1), :],
                        dst_v.at[pl.ds(CPW, 1), :])
        pltpu.async_copy(feat.at[src_v.at[CPW]], rows_v.at[0],
                         gsem[0]).wait()
        pltpu.sync_copy(rows_v.at[0], acc_s.at[dst_v.at[CPW]], add=True)
        if with_deg:
            pltpu.sync_copy(ones_v, dacc_s.at[dst_v.at[CPW]], add=True)

    plsc.subcore_barrier()

    pltpu.sync_copy(acc_s.at[pl.ds(s * RPT, RPT), :],
                    out.at[c, pl.ds(s * RPT, RPT), :])
    if with_deg:
        pltpu.sync_copy(dacc_s.at[pl.ds(s * RPT, RPT)],
                        deg_out.at[c, pl.ds(s * RPT, RPT)])


_SEMS = [pltpu.SemaphoreType.DMA] * (2 * NBUF)

_seg_deg = pl.kernel(
    functools.partial(_seg_body, True),
    out_type=[
        jax.ShapeDtypeStruct((NC, NP, HID), jnp.float32),
        jax.ShapeDtypeStruct((NC, NP), jnp.float32),
    ],
    mesh=plsc.VectorSubcoreMesh(core_axis_name="c", subcore_axis_name="s"),
    scratch_types=[
        pltpu.VMEM((CPW + 1, CHUNK), jnp.int32),
        pltpu.VMEM((CPW + 1, CHUNK), jnp.int32),
        pltpu.VMEM((NBUF, CHUNK, HID), jnp.float32),
        pltpu.VMEM((CHUNK,), jnp.float32),
        pltpu.VMEM_SHARED((NP, HID), jnp.float32),
        pltpu.VMEM_SHARED((NP,), jnp.float32),
        pltpu.SemaphoreType.DMA,
    ] + _SEMS,
    compiler_params=pltpu.CompilerParams(use_tc_tiling_on_sc=False),
)

_seg_nodeg = pl.kernel(
    functools.partial(_seg_body, False),
    out_type=jax.ShapeDtypeStruct((NC, NP, HID), jnp.float32),
    mesh=plsc.VectorSubcoreMesh(core_axis_name="c", subcore_axis_name="s"),
    scratch_types=[
        pltpu.VMEM((CPW + 1, CHUNK), jnp.int32),
        pltpu.VMEM((CPW + 1, CHUNK), jnp.int32),
        pltpu.VMEM((NBUF, CHUNK, HID), jnp.float32),
        pltpu.VMEM_SHARED((NP, HID), jnp.float32),
    ] + _SEMS,
    compiler_params=pltpu.CompilerParams(use_tc_tiling_on_sc=False),
)


def _comb1_body(ps_ref, pd_ref, z_ref, h_ref):
    ssum = ps_ref[0] + ps_ref[1]
    deg = pd_ref[0] + pd_ref[1]
    o = ssum / jnp.maximum(deg, 1.0) + z_ref[...]
    nrm = jnp.sqrt(jnp.sum(o * o, axis=-1, keepdims=True))
    h_ref[...] = jnp.maximum(o / jnp.maximum(nrm, 1e-12), 0.0)


def _comb1(ps, pd, z1):
    return pl.pallas_call(
        _comb1_body,
        grid=(N // BLK,),
        in_specs=[
            pl.BlockSpec((NC, BLK, HID), lambda i: (0, i, 0)),
            pl.BlockSpec((NC, BLK, 1), lambda i: (0, i, 0)),
            pl.BlockSpec((BLK, HID), lambda i: (i, 0)),
        ],
        out_specs=pl.BlockSpec((BLK, HID), lambda i: (i, 0)),
        out_shape=jax.ShapeDtypeStruct((N, HID), jnp.float32),
    )(ps, pd, z1)


def _comb2_body(ps_ref, pd_ref, h_ref, wl_ref, bl_ref, wr_ref, out_ref):
    ssum = ps_ref[0] + ps_ref[1]
    deg = pd_ref[0] + pd_ref[1]
    mean = ssum / jnp.maximum(deg, 1.0)
    o = (lax.dot_general(mean, wl_ref[...], (((1,), (1,)), ((), ())),
                         preferred_element_type=jnp.float32)
         + bl_ref[...]
         + lax.dot_general(h_ref[...], wr_ref[...], (((1,), (1,)), ((), ())),
                           preferred_element_type=jnp.float32))
    nrm = jnp.sqrt(jnp.sum(o * o, axis=-1, keepdims=True))
    out_ref[...] = o / jnp.maximum(nrm, 1e-12)


def _comb2(ps, pd, h, wl, bl, wr):
    return pl.pallas_call(
        _comb2_body,
        grid=(N // BLK,),
        in_specs=[
            pl.BlockSpec((NC, BLK, HID), lambda i: (0, i, 0)),
            pl.BlockSpec((NC, BLK, 1), lambda i: (0, i, 0)),
            pl.BlockSpec((BLK, HID), lambda i: (i, 0)),
            pl.BlockSpec(memory_space=pltpu.VMEM),
            pl.BlockSpec(memory_space=pltpu.VMEM),
            pl.BlockSpec(memory_space=pltpu.VMEM),
        ],
        out_specs=pl.BlockSpec((BLK, OUT), lambda i: (i, 0)),
        out_shape=jax.ShapeDtypeStruct((N, OUT), jnp.float32),
    )(ps, pd, h, wl, bl, wr)


def kernel(x, edge_index, W1l, b1l, W1r, W2l, b2l, W2r):
    eidx = edge_index.astype(jnp.int32).reshape(2, NCH, CHUNK)
    zrow = jnp.zeros((NP, HID), jnp.float32)
    zdeg = jnp.zeros((NP,), jnp.float32)

    y1, z1 = _proj(x, W1l, W1r, b1l.reshape(1, HID))
    ps1, pd1 = _seg_deg(y1, eidx, zrow, zdeg)
    pd1 = pd1.reshape(NC, NP, 1)
    h = _comb1(ps1, pd1, z1)
    ps2 = _seg_nodeg(h, eidx, zrow, zdeg)
    out = _comb2(ps2, pd1, h, W2l, b2l.reshape(1, OUT), W2r)
    return out


# speedup vs baseline: 1.0261x; 1.0261x over previous
import functools

import jax
import jax.numpy as jnp
from jax import lax
from jax.experimental import pallas as pl
from jax.experimental.pallas import tpu as pltpu
from jax.experimental.pallas import tpu_sc as plsc

N = 10000
E = 320000
INC = 128
HID = 32
OUT = 128

NC = 2
NS = 16
NW = NC * NS
CHUNK = 128
NCH = E // CHUNK
CPW = NCH // NW
NEXTRA = NCH - CPW * NW
NP = 10240
RPT = NP // NS

BLK = 5000


def _proj_body(x_ref, wl_ref, wr_ref, bl_ref, y_ref, z_ref):
    x = x_ref[...]
    y_ref[...] = lax.dot_general(x, wl_ref[...], (((1,), (1,)), ((), ())),
                                 preferred_element_type=jnp.float32)
    z_ref[...] = lax.dot_general(x, wr_ref[...], (((1,), (1,)), ((), ())),
                                 preferred_element_type=jnp.float32) + bl_ref[...]


def _proj(x, wl, wr, bl):
    return pl.pallas_call(
        _proj_body,
        grid=(N // BLK,),
        in_specs=[
            pl.BlockSpec((BLK, INC), lambda i: (i, 0)),
            pl.BlockSpec(memory_space=pltpu.VMEM),
            pl.BlockSpec(memory_space=pltpu.VMEM),
            pl.BlockSpec(memory_space=pltpu.VMEM),
        ],
        out_specs=[
            pl.BlockSpec((BLK, HID), lambda i: (i, 0)),
            pl.BlockSpec((BLK, HID), lambda i: (i, 0)),
        ],
        out_shape=[
            jax.ShapeDtypeStruct((N, HID), jnp.float32),
            jax.ShapeDtypeStruct((N, HID), jnp.float32),
        ],
    )(x, wl, wr, bl)


NBUF = 6


def _seg_body(with_deg, feat, eidx, zrow, zdeg, *refs):
    if with_deg:
        (out, deg_out, src_v, dst_v, rows_v, ones_v, acc_s, dacc_s,
         dsem, *sems) = refs
    else:
        out, src_v, dst_v, rows_v, acc_s, *sems = refs
    gsem = sems[:NBUF]
    ssem = sems[NBUF:]

    c = lax.axis_index("c")
    s = lax.axis_index("s")
    w = c * NS + s

    pltpu.sync_copy(zrow.at[pl.ds(s * RPT, RPT), :],
                    acc_s.at[pl.ds(s * RPT, RPT), :])
    if with_deg:
        pltpu.sync_copy(zdeg.at[pl.ds(s * RPT, RPT)],
                        dacc_s.at[pl.ds(s * RPT, RPT)])
        for i in range(CHUNK // 16):
            ones_v[pl.ds(i * 16, 16)] = jnp.ones((16,), jnp.float32)

    plsc.subcore_barrier()

    base = CPW * w + jnp.minimum(w, NEXTRA)

    pltpu.sync_copy(eidx.at[0, pl.ds(base, CPW), :],
                    src_v.at[pl.ds(0, CPW), :])
    pltpu.sync_copy(eidx.at[1, pl.ds(base, CPW), :],
                    dst_v.at[pl.ds(0, CPW), :])

    for b in range(NBUF):
        pltpu.async_copy(feat.at[src_v.at[b]], rows_v.at[b], gsem[b])

    niter = CPW // NBUF

    @pl.loop(0, niter)
    def _grp(g):
        jb = g * NBUF
        degs = []
        for b in range(NBUF):
            j = jb + b
            pltpu.make_async_copy(feat.at[src_v.at[j]], rows_v.at[b],
                                  gsem[b]).wait()
            sc = pltpu.async_copy(rows_v.at[b], acc_s.at[dst_v.at[j]],
                                  ssem[b], add=True)
            if with_deg:
                degs.append(pltpu.async_copy(ones_v,
                                             dacc_s.at[dst_v.at[j]],
                                             dsem, add=True))
            sc.wait()
            @pl.when(g < niter - 1)
            def _refill():
                pltpu.async_copy(feat.at[src_v.at[j + NBUF]],
                                 rows_v.at[b], gsem[b])
        for d in degs:
            d.wait()

    @pl.when(w < NEXTRA)
    def _extra():
        pltpu.sync_copy(eidx.at[0, pl.ds(base + CPW, 1), :],
                        src_v.at[pl.ds(CPW, 1), :])
        pltpu.sync_copy(eidx.at[1, pl.ds(base + CPW, 1), :],
                        dst_v.at[pl.ds(CPW, 1), :])
        pltpu.async_copy(feat.at[src_v.at[CPW]], rows_v.at[0],
                         gsem[0]).wait()
        pltpu.sync_copy(rows_v.at[0], acc_s.at[dst_v.at[CPW]], add=True)
        if with_deg:
            pltpu.sync_copy(ones_v, dacc_s.at[dst_v.at[CPW]], add=True)

    plsc.subcore_barrier()

    pltpu.sync_copy(acc_s.at[pl.ds(s * RPT, RPT), :],
                    out.at[c, pl.ds(s * RPT, RPT), :])
    if with_deg:
        pltpu.sync_copy(dacc_s.at[pl.ds(s * RPT, RPT)],
                        deg_out.at[c, pl.ds(s * RPT, RPT)])


_SEMS = [pltpu.SemaphoreType.DMA] * (2 * NBUF)

_seg_deg = pl.kernel(
    functools.partial(_seg_body, True),
    out_type=[
        jax.ShapeDtypeStruct((NC, NP, HID), jnp.float32),
        jax.ShapeDtypeStruct((NC, NP), jnp.float32),
    ],
    mesh=plsc.VectorSubcoreMesh(core_axis_name="c", subcore_axis_name="s"),
    scratch_types=[
        pltpu.VMEM((CPW + 1, CHUNK), jnp.int32),
        pltpu.VMEM((CPW + 1, CHUNK), jnp.int32),
        pltpu.VMEM((NBUF, CHUNK, HID), jnp.float32),
        pltpu.VMEM((CHUNK,), jnp.float32),
        pltpu.VMEM_SHARED((NP, HID), jnp.float32),
        pltpu.VMEM_SHARED((NP,), jnp.float32),
        pltpu.SemaphoreType.DMA,
    ] + _SEMS,
    compiler_params=pltpu.CompilerParams(use_tc_tiling_on_sc=False),
)

_seg_nodeg = pl.kernel(
    functools.partial(_seg_body, False),
    out_type=jax.ShapeDtypeStruct((NC, NP, HID), jnp.float32),
    mesh=plsc.VectorSubcoreMesh(core_axis_name="c", subcore_axis_name="s"),
    scratch_types=[
        pltpu.VMEM((CPW + 1, CHUNK), jnp.int32),
        pltpu.VMEM((CPW + 1, CHUNK), jnp.int32),
        pltpu.VMEM((NBUF, CHUNK, HID), jnp.float32),
        pltpu.VMEM_SHARED((NP, HID), jnp.float32),
    ] + _SEMS,
    compiler_params=pltpu.CompilerParams(use_tc_tiling_on_sc=False),
)


def _comb1_body(ps_ref, pd_ref, z_ref, h_ref):
    ssum = ps_ref[0] + ps_ref[1]
    deg = pd_ref[0] + pd_ref[1]
    o = ssum / jnp.maximum(deg, 1.0) + z_ref[...]
    nrm = jnp.sqrt(jnp.sum(o * o, axis=-1, keepdims=True))
    h_ref[...] = jnp.maximum(o / jnp.maximum(nrm, 1e-12), 0.0)


def _comb1(ps, pd, z1):
    return pl.pallas_call(
        _comb1_body,
        grid=(N // BLK,),
        in_specs=[
            pl.BlockSpec((NC, BLK, HID), lambda i: (0, i, 0)),
            pl.BlockSpec((NC, BLK, 1), lambda i: (0, i, 0)),
            pl.BlockSpec((BLK, HID), lambda i: (i, 0)),
        ],
        out_specs=pl.BlockSpec((BLK, HID), lambda i: (i, 0)),
        out_shape=jax.ShapeDtypeStruct((N, HID), jnp.float32),
    )(ps, pd, z1)


def _comb2_body(ps_ref, pd_ref, h_ref, wl_ref, bl_ref, wr_ref, out_ref):
    ssum = ps_ref[0] + ps_ref[1]
    deg = pd_ref[0] + pd_ref[1]
    mean = ssum / jnp.maximum(deg, 1.0)
    o = (lax.dot_general(mean, wl_ref[...], (((1,), (1,)), ((), ())),
                         preferred_element_type=jnp.float32)
         + bl_ref[...]
         + lax.dot_general(h_ref[...], wr_ref[...], (((1,), (1,)), ((), ())),
                           preferred_element_type=jnp.float32))
    nrm = jnp.sqrt(jnp.sum(o * o, axis=-1, keepdims=True))
    out_ref[...] = o / jnp.maximum(nrm, 1e-12)


def _comb2(ps, pd, h, wl, bl, wr):
    return pl.pallas_call(
        _comb2_body,
        grid=(N // BLK,),
        in_specs=[
            pl.BlockSpec((NC, BLK, HID), lambda i: (0, i, 0)),
            pl.BlockSpec((NC, BLK, 1), lambda i: (0, i, 0)),
            pl.BlockSpec((BLK, HID), lambda i: (i, 0)),
            pl.BlockSpec(memory_space=pltpu.VMEM),
            pl.BlockSpec(memory_space=pltpu.VMEM),
            pl.BlockSpec(memory_space=pltpu.VMEM),
        ],
        out_specs=pl.BlockSpec((BLK, OUT), lambda i: (i, 0)),
        out_shape=jax.ShapeDtypeStruct((N, OUT), jnp.float32),
    )(ps, pd, h, wl, bl, wr)


def kernel(x, edge_index, W1l, b1l, W1r, W2l, b2l, W2r):
    eidx = edge_index.astype(jnp.int32).reshape(2, NCH, CHUNK)
    zrow = jnp.zeros((NP, HID), jnp.float32)
    zdeg = jnp.zeros((NP,), jnp.float32)

    y1, z1 = _proj(x, W1l, W1r, b1l.reshape(1, HID))
    ps1, pd1 = _seg_deg(y1, eidx, zrow, zdeg)
    pd1 = pd1.reshape(NC, NP, 1)
    h = _comb1(ps1, pd1, z1)
    ps2 = _seg_nodeg(h, eidx, zrow, zdeg)
    out = _comb2(ps2, pd1, h, W2l, b2l.reshape(1, OUT), W2r)
    return out
